# baseline (speedup 1.0000x reference)
import functools

import jax
import jax.numpy as jnp
from jax import lax
from jax.experimental import pallas as pl
from jax.experimental.pallas import tpu as pltpu
from jax.experimental.pallas import tpu_sc as plsc

N_A = 131072
N_P = 32768
B = 512
D = 300

NC = 2
NS = 16
NW = NC * NS

G = 128
DP = 384
A_PER_W = N_A // NW
F_PER_W = N_P // NW
A_GRPS = A_PER_W // G
F_GRPS = F_PER_W // G
ROWS_PER_TILE = B // NS


def _tp_body(in_ref, out_ref):
    a = in_ref[...]
    at = jnp.transpose(a, (1, 0))
    padded = jnp.pad(at, ((0, 0), (0, DP - D)))
    out_ref[...] = padded.reshape(G, 3, 128).reshape(G * 3, 128)


def _transpose_pad(x_t, n):
    return pl.pallas_call(
        _tp_body,
        grid=(n // G,),
        in_specs=[pl.BlockSpec((D, G), lambda i: (0, i))],
        out_specs=pl.BlockSpec((G * 3, 128), lambda i: (i, 0)),
        out_shape=jax.ShapeDtypeStruct((n * 3, 128), jnp.float32),
    )(x_t)


def _sc_body(atom_h, frag_h, aseg_h, fseg_h, z2d_h, z1_h,
             part_o, cnt_o,
             acc, cnta, cntf, buf, aidx, fidx, ones, zc,
             sem0, sem1):
    c = lax.axis_index("c")
    s = lax.axis_index("s")
    wid = s * NC + c

    for i in range(G // 16):
        ones[pl.ds(i * 16, 16)] = jnp.ones((16,), jnp.float32)

    r0 = s * ROWS_PER_TILE
    pltpu.sync_copy(z2d_h.at[pl.ds(r0, ROWS_PER_TILE)],
                    buf.at[0, pl.ds(0, ROWS_PER_TILE)])
    pltpu.sync_copy(buf.at[0, pl.ds(0, ROWS_PER_TILE)],
                    acc.at[pl.ds(r0, ROWS_PER_TILE)])
    pltpu.sync_copy(z1_h.at[pl.ds(r0, ROWS_PER_TILE)], zc)
    pltpu.sync_copy(zc, cnta.at[pl.ds(r0, ROWS_PER_TILE)])
    pltpu.sync_copy(zc, cntf.at[pl.ds(r0, ROWS_PER_TILE)])
    plsc.subcore_barrier()

    pltpu.sync_copy(aseg_h.at[pl.ds(wid * A_GRPS, A_GRPS)], aidx)
    pltpu.sync_copy(fseg_h.at[pl.ds(wid * F_GRPS, F_GRPS)], fidx)

    sems = (sem0, sem1)

    def run(src_h, idx, n_grps, base, cnt_ref):
        pltpu.async_copy(src_h.at[pl.ds(base, G)], buf.at[0], sems[0])
        for g in range(n_grps):
            cur = g & 1
            if g + 1 < n_grps:
                pltpu.async_copy(src_h.at[pl.ds(base + (g + 1) * G, G)],
                                 buf.at[1 - cur], sems[1 - cur])
            pltpu.make_async_copy(src_h.at[pl.ds(base + g * G, G)],
                                  buf.at[cur], sems[cur]).wait()
            pltpu.sync_copy(buf.at[cur], acc.at[idx.at[g]], add=True)
            pltpu.sync_copy(ones, cnt_ref.at[idx.at[g]], add=True)

    run(atom_h, aidx, A_GRPS, wid * A_PER_W, cnta)
    run(frag_h, fidx, F_GRPS, wid * F_PER_W, cntf)

    plsc.subcore_barrier()

    pltpu.sync_copy(acc.at[pl.ds(r0, ROWS_PER_TILE)],
                    buf.at[0, pl.ds(0, ROWS_PER_TILE)])
    pltpu.sync_copy(buf.at[0, pl.ds(0, ROWS_PER_TILE)],
                    part_o.at[c].at[pl.ds(r0, ROWS_PER_TILE)])
    pltpu.sync_copy(cnta.at[pl.ds(r0, ROWS_PER_TILE)], zc)
    pltpu.sync_copy(zc, cnt_o.at[c, 0].at[pl.ds(r0, ROWS_PER_TILE)])
    pltpu.sync_copy(cntf.at[pl.ds(r0, ROWS_PER_TILE)], zc)
    pltpu.sync_copy(zc, cnt_o.at[c, 1].at[pl.ds(r0, ROWS_PER_TILE)])


_sc_call = pl.kernel(
    _sc_body,
    out_type=(
        jax.ShapeDtypeStruct((NC, B, DP), jnp.float32),
        jax.ShapeDtypeStruct((NC, 2, B), jnp.float32),
    ),
    mesh=plsc.VectorSubcoreMesh(core_axis_name="c", subcore_axis_name="s"),
    compiler_params=pltpu.CompilerParams(use_tc_tiling_on_sc=False),
    scratch_types=[
        pltpu.VMEM_SHARED((B, DP), jnp.float32),
        pltpu.VMEM_SHARED((B,), jnp.float32),
        pltpu.VMEM_SHARED((B,), jnp.float32),
        pltpu.VMEM((2, G, DP), jnp.float32),
        pltpu.VMEM((A_GRPS, G), jnp.int32),
        pltpu.VMEM((F_GRPS, G), jnp.int32),
        pltpu.VMEM((G,), jnp.float32),
        pltpu.VMEM((ROWS_PER_TILE,), jnp.float32),
        pltpu.SemaphoreType.DMA,
        pltpu.SemaphoreType.DMA,
    ],
)


def _combine_body(part_ref, cnt_ref, out_ref):
    p = part_ref[0, :, :D] + part_ref[1, :, :D]
    cs = cnt_ref[0] + cnt_ref[1]
    ma = jnp.max(cs[0:1, :])
    mf = jnp.max(cs[1:2, :])
    out_ref[...] = p * (1.0 / (ma + mf))


def kernel(atom_embed, frag_embed, atom_seg, frag_seg):
    atom_flat = _transpose_pad(atom_embed.T, N_A)
    frag_flat = _transpose_pad(frag_embed.T, N_P)
    atom_p = atom_flat.reshape(N_A, DP)
    frag_p = frag_flat.reshape(N_P, DP)
    aseg = atom_seg.astype(jnp.int32).reshape(N_A // G, G)
    fseg = frag_seg.astype(jnp.int32).reshape(N_P // G, G)
    z2d = jnp.zeros((B, DP), jnp.float32)
    z1 = jnp.zeros((B,), jnp.float32)
    part, cnt = _sc_call(atom_p, frag_p, aseg, fseg, z2d, z1)
    return pl.pallas_call(
        _combine_body,
        out_shape=jax.ShapeDtypeStruct((B, D), jnp.float32),
    )(part, cnt)


# speedup vs baseline: 2.2191x; 2.2191x over previous
import functools

import jax
import jax.numpy as jnp
from jax import lax
from jax.experimental import pallas as pl
from jax.experimental.pallas import tpu as pltpu
from jax.experimental.pallas import tpu_sc as plsc

N_A = 131072
N_P = 32768
B = 512
D = 300

NC = 2
NS = 16
NW = NC * NS

G = 128
W = 1024
A_PER_W = N_A // NW
F_PER_W = N_P // NW
A_GRPS = A_PER_W // G
F_GRPS = F_PER_W // G
ROWS_PER_TILE = B // NS


def _tp_body(in_ref, o0_ref, o1_ref, o2_ref):
    a = in_ref[...]
    o0_ref[...] = jnp.transpose(a[0:128, :], (1, 0))
    o1_ref[...] = jnp.transpose(a[128:256, :], (1, 0))
    t2 = jnp.transpose(a[256:300, :], (1, 0))
    o2_ref[...] = jnp.pad(t2, ((0, 0), (0, 128 - (D - 256))))


def _transpose_panels(x_t, n):
    return pl.pallas_call(
        _tp_body,
        grid=(n // W,),
        in_specs=[pl.BlockSpec((D, W), lambda i: (0, i))],
        out_specs=[pl.BlockSpec((W, 128), lambda i: (i, 0))] * 3,
        out_shape=[jax.ShapeDtypeStruct((n, 128), jnp.float32)] * 3,
    )(x_t)


def _sc_body(a0, a1, a2, f0, f1, f2, aseg_h, fseg_h, z2d_h, z1_h,
             part_o, cnt_o,
             acc0, acc1, acc2, cnta, cntf, buf, aidx, fidx, ones, zc,
             sem0, sem1):
    c = lax.axis_index("c")
    s = lax.axis_index("s")
    wid = s * NC + c
    accs = (acc0, acc1, acc2)

    for i in range(G // 16):
        ones[pl.ds(i * 16, 16)] = jnp.ones((16,), jnp.float32)

    r0 = s * ROWS_PER_TILE
    pltpu.sync_copy(z2d_h.at[pl.ds(r0, ROWS_PER_TILE)],
                    buf.at[0, 0, pl.ds(0, ROWS_PER_TILE)])
    for m in range(3):
        pltpu.sync_copy(buf.at[0, 0, pl.ds(0, ROWS_PER_TILE)],
                        accs[m].at[pl.ds(r0, ROWS_PER_TILE)])
    pltpu.sync_copy(z1_h.at[pl.ds(r0, ROWS_PER_TILE)], zc)
    pltpu.sync_copy(zc, cnta.at[pl.ds(r0, ROWS_PER_TILE)])
    pltpu.sync_copy(zc, cntf.at[pl.ds(r0, ROWS_PER_TILE)])
    plsc.subcore_barrier()

    pltpu.sync_copy(aseg_h.at[pl.ds(wid * A_GRPS, A_GRPS)], aidx)
    pltpu.sync_copy(fseg_h.at[pl.ds(wid * F_GRPS, F_GRPS)], fidx)

    sems = (sem0, sem1)

    def run(srcs, idx, n_grps, base, cnt_ref):
        def start(g, slot):
            for m in range(3):
                pltpu.async_copy(srcs[m].at[pl.ds(base + g * G, G)],
                                 buf.at[slot, m], sems[slot])

        def wait(g, slot):
            for m in range(3):
                pltpu.make_async_copy(srcs[m].at[pl.ds(base + g * G, G)],
                                      buf.at[slot, m], sems[slot]).wait()

        start(0, 0)
        for g in range(n_grps):
            cur = g & 1
            if g + 1 < n_grps:
                start(g + 1, 1 - cur)
            wait(g, cur)
            for m in range(3):
                pltpu.sync_copy(buf.at[cur, m], accs[m].at[idx.at[g]],
                                add=True)
            pltpu.sync_copy(ones, cnt_ref.at[idx.at[g]], add=True)

    run((a0, a1, a2), aidx, A_GRPS, wid * A_PER_W, cnta)
    run((f0, f1, f2), fidx, F_GRPS, wid * F_PER_W, cntf)

    plsc.subcore_barrier()

    for m in range(3):
        pltpu.sync_copy(accs[m].at[pl.ds(r0, ROWS_PER_TILE)],
                        buf.at[0, m, pl.ds(0, ROWS_PER_TILE)])
        pltpu.sync_copy(buf.at[0, m, pl.ds(0, ROWS_PER_TILE)],
                        part_o.at[c, m].at[pl.ds(r0, ROWS_PER_TILE)])
    pltpu.sync_copy(cnta.at[pl.ds(r0, ROWS_PER_TILE)], zc)
    pltpu.sync_copy(zc, cnt_o.at[c, 0].at[pl.ds(r0, ROWS_PER_TILE)])
    pltpu.sync_copy(cntf.at[pl.ds(r0, ROWS_PER_TILE)], zc)
    pltpu.sync_copy(zc, cnt_o.at[c, 1].at[pl.ds(r0, ROWS_PER_TILE)])


_sc_call = pl.kernel(
    _sc_body,
    out_type=(
        jax.ShapeDtypeStruct((NC, 3, B, 128), jnp.float32),
        jax.ShapeDtypeStruct((NC, 2, B), jnp.float32),
    ),
    mesh=plsc.VectorSubcoreMesh(core_axis_name="c", subcore_axis_name="s"),
    compiler_params=pltpu.CompilerParams(use_tc_tiling_on_sc=False),
    scratch_types=[
        pltpu.VMEM_SHARED((B, 128), jnp.float32),
        pltpu.VMEM_SHARED((B, 128), jnp.float32),
        pltpu.VMEM_SHARED((B, 128), jnp.float32),
        pltpu.VMEM_SHARED((B,), jnp.float32),
        pltpu.VMEM_SHARED((B,), jnp.float32),
        pltpu.VMEM((2, 3, G, 128), jnp.float32),
        pltpu.VMEM((A_GRPS, G), jnp.int32),
        pltpu.VMEM((F_GRPS, G), jnp.int32),
        pltpu.VMEM((G,), jnp.float32),
        pltpu.VMEM((ROWS_PER_TILE,), jnp.float32),
        pltpu.SemaphoreType.DMA,
        pltpu.SemaphoreType.DMA,
    ],
)


def _combine_body(part_ref, cnt_ref, out_ref):
    p = part_ref[0] + part_ref[1]
    cs = cnt_ref[0] + cnt_ref[1]
    ma = jnp.max(cs[0:1, :])
    mf = jnp.max(cs[1:2, :])
    scale = 1.0 / (ma + mf)
    out_ref[:, 0:128] = p[0] * scale
    out_ref[:, 128:256] = p[1] * scale
    out_ref[:, 256:300] = p[2, :, 0:44] * scale


def kernel(atom_embed, frag_embed, atom_seg, frag_seg):
    a0, a1, a2 = _transpose_panels(atom_embed.T, N_A)
    f0, f1, f2 = _transpose_panels(frag_embed.T, N_P)
    aseg = atom_seg.astype(jnp.int32).reshape(N_A // G, G)
    fseg = frag_seg.astype(jnp.int32).reshape(N_P // G, G)
    z2d = jnp.zeros((B, 128), jnp.float32)
    z1 = jnp.zeros((B,), jnp.float32)
    part, cnt = _sc_call(a0, a1, a2, f0, f1, f2, aseg, fseg, z2d, z1)
    return pl.pallas_call(
        _combine_body,
        out_shape=jax.ShapeDtypeStruct((B, D), jnp.float32),
    )(part, cnt)


# speedup vs baseline: 3.0155x; 1.3589x over previous
import functools

import jax
import jax.numpy as jnp
from jax import lax
from jax.experimental import pallas as pl
from jax.experimental.pallas import tpu as pltpu
from jax.experimental.pallas import tpu_sc as plsc

N_A = 131072
N_P = 32768
B = 512
D = 300

NC = 2
NS = 16
NW = NC * NS

G = 128
W = 1024
DP = 384
CH = 32768
C_PER_W = CH // NW
C_GRPS = C_PER_W // G
ROWS_PER_TILE = B // NS


def _tp_body(in_ref, o0_ref, o1_ref, o2_ref):
    a = in_ref[...]
    o0_ref[...] = jnp.transpose(a[0:128, :], (1, 0))
    o1_ref[...] = jnp.transpose(a[128:256, :], (1, 0))
    t2 = jnp.transpose(a[256:300, :], (1, 0))
    o2_ref[...] = jnp.pad(t2, ((0, 0), (0, 128 - (D - 256))))


def _transpose_panels(x_t, chunk):
    base = chunk * (CH // W)
    return pl.pallas_call(
        _tp_body,
        grid=(CH // W,),
        in_specs=[pl.BlockSpec((D, W), lambda i: (0, i + base))],
        out_specs=[pl.BlockSpec((W, 128), lambda i: (i, 0))] * 3,
        out_shape=[jax.ShapeDtypeStruct((CH, 128), jnp.float32)] * 3,
    )(x_t)


def _sc_body(x0, x1, x2, seg_h, z2d_h, z1_h,
             part_o, cnt_o,
             acc, cnt, buf, idx, ones, zc,
             sem0, sem1):
    c = lax.axis_index("c")
    s = lax.axis_index("s")
    wid = s * NC + c
    srcs = (x0, x1, x2)

    for i in range(G // 16):
        ones[pl.ds(i * 16, 16)] = jnp.ones((16,), jnp.float32)

    r0 = s * ROWS_PER_TILE
    pltpu.sync_copy(z2d_h.at[pl.ds(r0, ROWS_PER_TILE)],
                    buf.at[0, pl.ds(0, ROWS_PER_TILE)])
    pltpu.sync_copy(buf.at[0, pl.ds(0, ROWS_PER_TILE)],
                    acc.at[pl.ds(r0, ROWS_PER_TILE)])
    pltpu.sync_copy(z1_h.at[pl.ds(r0, ROWS_PER_TILE)], zc)
    pltpu.sync_copy(zc, cnt.at[pl.ds(r0, ROWS_PER_TILE)])
    plsc.subcore_barrier()

    pltpu.sync_copy(seg_h.at[pl.ds(wid * C_GRPS, C_GRPS)], idx)

    sems = (sem0, sem1)
    base = wid * C_PER_W

    def start(g, slot):
        for m in range(3):
            pltpu.async_copy(srcs[m].at[pl.ds(base + g * G, G)],
                             buf.at[slot, :, pl.ds(128 * m, 128)],
                             sems[slot])

    def wait(g, slot):
        for m in range(3):
            pltpu.make_async_copy(srcs[m].at[pl.ds(base + g * G, G)],
                                  buf.at[slot, :, pl.ds(128 * m, 128)],
                                  sems[slot]).wait()

    start(0, 0)
    for g in range(C_GRPS):
        cur = g & 1
        if g + 1 < C_GRPS:
            start(g + 1, 1 - cur)
        wait(g, cur)
        pltpu.sync_copy(buf.at[cur], acc.at[idx.at[g]], add=True)
        pltpu.sync_copy(ones, cnt.at[idx.at[g]], add=True)

    plsc.subcore_barrier()

    pltpu.sync_copy(acc.at[pl.ds(r0, ROWS_PER_TILE)],
                    buf.at[0, pl.ds(0, ROWS_PER_TILE)])
    pltpu.sync_copy(buf.at[0, pl.ds(0, ROWS_PER_TILE)],
                    part_o.at[c].at[pl.ds(r0, ROWS_PER_TILE)])
    pltpu.sync_copy(cnt.at[pl.ds(r0, ROWS_PER_TILE)], zc)
    pltpu.sync_copy(zc, cnt_o.at[c].at[pl.ds(r0, ROWS_PER_TILE)])


_sc_chunk = pl.kernel(
    _sc_body,
    out_type=(
        jax.ShapeDtypeStruct((NC, B, DP), jnp.float32),
        jax.ShapeDtypeStruct((NC, B), jnp.float32),
    ),
    mesh=plsc.VectorSubcoreMesh(core_axis_name="c", subcore_axis_name="s"),
    compiler_params=pltpu.CompilerParams(use_tc_tiling_on_sc=False),
    scratch_types=[
        pltpu.VMEM_SHARED((B, DP), jnp.float32),
        pltpu.VMEM_SHARED((B,), jnp.float32),
        pltpu.VMEM((2, G, DP), jnp.float32),
        pltpu.VMEM((C_GRPS, G), jnp.int32),
        pltpu.VMEM((G,), jnp.float32),
        pltpu.VMEM((ROWS_PER_TILE,), jnp.float32),
        pltpu.SemaphoreType.DMA,
        pltpu.SemaphoreType.DMA,
    ],
)


def _combine_body(p0, p1, p2, p3, p4, c0, c1, c2, c3, c4, out_ref):
    p = (p0[0, :, :D] + p0[1, :, :D] + p1[0, :, :D] + p1[1, :, :D]
         + p2[0, :, :D] + p2[1, :, :D] + p3[0, :, :D] + p3[1, :, :D]
         + p4[0, :, :D] + p4[1, :, :D])
    ca = (c0[0:1] + c0[1:2] + c1[0:1] + c1[1:2] + c2[0:1] + c2[1:2]
          + c3[0:1] + c3[1:2])
    cf = c4[0:1] + c4[1:2]
    out_ref[...] = p * (1.0 / (jnp.max(ca) + jnp.max(cf)))


def kernel(atom_embed, frag_embed, atom_seg, frag_seg):
    aseg = atom_seg.astype(jnp.int32).reshape(N_A // G, G)
    fseg = frag_seg.astype(jnp.int32).reshape(N_P // G, G)
    z2d = jnp.zeros((B, DP), jnp.float32)
    z1 = jnp.zeros((B,), jnp.float32)

    atom_t = atom_embed.T
    parts, cnts = [], []
    for ch in range(4):
        x0, x1, x2 = _transpose_panels(atom_t, ch)
        seg = lax.dynamic_slice_in_dim(aseg, ch * (CH // G), CH // G)
        part, cnt = _sc_chunk(x0, x1, x2, seg, z2d, z1)
        parts.append(part)
        cnts.append(cnt)
    x0, x1, x2 = _transpose_panels(frag_embed.T, 0)
    part, cnt = _sc_chunk(x0, x1, x2, fseg, z2d, z1)
    parts.append(part)
    cnts.append(cnt)

    return pl.pallas_call(
        _combine_body,
        out_shape=jax.ShapeDtypeStruct((B, D), jnp.float32),
    )(*parts, *cnts)
